# Initial kernel scaffold; baseline (speedup 1.0000x reference)
import jax
import jax.numpy as jnp
from jax.experimental import pallas as pl


def kernel(x, batch_indices, sonata_features, sonata_batch_indices, temperature, W_x, b_x, W_fx, b_fx, W_slice, b_slice, W_q, b_q, W_k, b_k, W_v, b_v, W_ck, b_ck, W_cv, b_cv, W_out):
    raise NotImplementedError("write your pallas kernel here")


# speedup vs baseline: 15.8082x; 15.8082x over previous
import jax
import jax.numpy as jnp
from jax.experimental import pallas as pl

DIM = 256
HEADS = 8
DIM_HEAD = 64
SLICE_NUM = 32
INNER = HEADS * DIM_HEAD
N_TOTAL = 4096
M_TOTAL = 1024
NB = 4
SCALE = DIM_HEAD ** (-0.5)
BG = NB * SLICE_NUM


def _softmax_rows(z):
    z = z - jnp.max(z, axis=1, keepdims=True)
    e = jnp.exp(z)
    return e / jnp.sum(e, axis=1, keepdims=True)


def _body(x_ref, bi_ref, son_ref, sbi_ref,
          Wx_ref, bx_ref, Wfx_ref, bfx_ref, Wsl_ref, bsl_ref,
          Wq_ref, bq_ref, Wk_ref, bk_ref, Wv_ref, bv_ref,
          Wck_ref, bck_ref, Wcv_ref, bcv_ref, Wout_ref, out_ref):
    f32 = jnp.float32
    x = x_ref[:, :]
    xm = jnp.dot(x, Wx_ref[:, :], preferred_element_type=f32) + bx_ref[:, :]
    fxm = jnp.dot(x, Wfx_ref[:, :], preferred_element_type=f32) + bfx_ref[:, :]

    bi = bi_ref[:, :]
    blk = jax.lax.broadcasted_iota(jnp.int32, (N_TOTAL, BG), 1) // SLICE_NUM
    maskP = (blk == bi).astype(f32)

    sbi_row = sbi_ref[:, :]
    son = son_ref[:, :]
    ones_col = jnp.ones((N_TOTAL, 1), f32)

    out_acc = jnp.zeros((N_TOTAL, DIM), f32)
    for h in range(HEADS):
        c0, c1 = h * DIM_HEAD, (h + 1) * DIM_HEAD
        xm_h = xm[:, c0:c1]
        fx_h = fxm[:, c0:c1]
        logits = jnp.dot(xm_h, Wsl_ref[h], preferred_element_type=f32) + bsl_ref[h]
        sw = _softmax_rows(logits)
        swm = jnp.concatenate([sw, sw, sw, sw], axis=1) * maskP

        st = jax.lax.dot_general(swm, fx_h, (((0,), (0,)), ((), ())),
                                 preferred_element_type=f32)
        sn = jax.lax.dot_general(swm, ones_col, (((0,), (0,)), ((), ())),
                                 preferred_element_type=f32)
        stok = st / (sn + 1e-5)

        q = jnp.dot(stok, Wq_ref[:, :], preferred_element_type=f32) + bq_ref[:, :]
        k = jnp.dot(stok, Wk_ref[:, :], preferred_element_type=f32) + bk_ref[:, :]
        v = jnp.dot(stok, Wv_ref[:, :], preferred_element_type=f32) + bv_ref[:, :]
        k_son = jnp.dot(son[:, c0:c1], Wck_ref[:, :], preferred_element_type=f32) + bck_ref[:, :]
        v_son = jnp.dot(son[:, c0:c1], Wcv_ref[:, :], preferred_element_type=f32) + bcv_ref[:, :]

        blocks = []
        for b in range(NB):
            r0, r1 = b * SLICE_NUM, (b + 1) * SLICE_NUM
            qb, kb, vb = q[r0:r1], k[r0:r1], v[r0:r1]
            dots = jax.lax.dot_general(qb, kb, (((1,), (1,)), ((), ())),
                                       preferred_element_type=f32) * SCALE
            attn = _softmax_rows(dots)
            osb = jnp.dot(attn, vb, preferred_element_type=f32)

            m_row = (sbi_row == b).astype(f32)
            scores = jax.lax.dot_general(osb, k_son, (((1,), (1,)), ((), ())),
                                         preferred_element_type=f32) * SCALE
            scores = scores - (1.0 - m_row) * 1e9
            aw = _softmax_rows(scores)
            cross = jnp.dot(aw, v_son, preferred_element_type=f32)
            nonempty = (jnp.sum(m_row, keepdims=True) > 0.0).astype(f32)
            blocks.append(osb + cross * nonempty)
        full_os = jnp.concatenate(blocks, axis=0)

        out_x_h = jnp.dot(swm, full_os, preferred_element_type=f32)
        out_acc = out_acc + jnp.dot(out_x_h, Wout_ref[c0:c1, :],
                                    preferred_element_type=f32)
    out_ref[:, :] = out_acc


@jax.jit
def kernel(x, batch_indices, sonata_features, sonata_batch_indices, temperature,
           W_x, b_x, W_fx, b_fx, W_slice, b_slice, W_q, b_q, W_k, b_k,
           W_v, b_v, W_ck, b_ck, W_cv, b_cv, W_out):
    inv_t = 1.0 / temperature.reshape(HEADS)
    Wsl = W_slice[None, :, :] * inv_t[:, None, None]
    bsl = (b_slice[None, :] * inv_t[:, None])[:, None, :]
    bi = batch_indices.reshape(N_TOTAL, 1).astype(jnp.int32)
    sbi = sonata_batch_indices.reshape(1, M_TOTAL).astype(jnp.int32)
    return pl.pallas_call(
        _body,
        out_shape=jax.ShapeDtypeStruct((N_TOTAL, DIM), jnp.float32),
    )(x, bi, sonata_features, sbi,
      W_x, b_x.reshape(1, INNER), W_fx, b_fx.reshape(1, INNER),
      Wsl, bsl,
      W_q, b_q.reshape(1, DIM_HEAD), W_k, b_k.reshape(1, DIM_HEAD),
      W_v, b_v.reshape(1, DIM_HEAD), W_ck, b_ck.reshape(1, DIM_HEAD),
      W_cv, b_cv.reshape(1, DIM_HEAD), W_out)


# speedup vs baseline: 17.5891x; 1.1127x over previous
import jax
import jax.numpy as jnp
from jax.experimental import pallas as pl

DIM = 256
HEADS = 8
DIM_HEAD = 64
SLICE_NUM = 32
INNER = HEADS * DIM_HEAD
N_TOTAL = 4096
M_TOTAL = 1024
NB = 4
SCALE = DIM_HEAD ** (-0.5)
BG = NB * SLICE_NUM
HG = HEADS * SLICE_NUM


def _softmax_rows(z):
    z = z - jnp.max(z, axis=1, keepdims=True)
    e = jnp.exp(z)
    return e / jnp.sum(e, axis=1, keepdims=True)


def _body(x_ref, bi_ref, son_ref, sbi_ref,
          Wx_ref, bx_ref, Wslbd_ref, bsl_ref, Wfx_ref, bfx_ref,
          Wq_ref, bq_ref, Wk_ref, bk_ref, Wv_ref, bv_ref,
          Wck_ref, bck_ref, Wcv_ref, bcv_ref, Wout_ref, out_ref):
    f32 = jnp.float32
    bf = jnp.bfloat16
    x = x_ref[:, :]
    xb = x.astype(bf)

    Wxsl = jnp.dot(Wx_ref[:, :], Wslbd_ref[:, :], preferred_element_type=f32)
    bxsl = jnp.dot(bx_ref[:, :], Wslbd_ref[:, :], preferred_element_type=f32) + bsl_ref[:, :]
    logits_all = jnp.dot(xb, Wxsl.astype(bf), preferred_element_type=f32) + bxsl

    fxm = jnp.dot(xb, Wfx_ref[:, :].astype(bf), preferred_element_type=f32) + bfx_ref[:, :]
    fxm = fxm.astype(bf)

    bi = bi_ref[:, :]
    blk = jax.lax.broadcasted_iota(jnp.int32, (N_TOTAL, BG), 1) // SLICE_NUM
    maskP = (blk == bi).astype(f32)

    sbi_row = sbi_ref[:, :]
    son = son_ref[:, :]
    ones_col = jnp.ones((N_TOTAL, 1), bf)

    outx_cols = []
    for h in range(HEADS):
        c0, c1 = h * DIM_HEAD, (h + 1) * DIM_HEAD
        fx_h = fxm[:, c0:c1]
        sw = _softmax_rows(logits_all[:, h * SLICE_NUM:(h + 1) * SLICE_NUM])
        swm = (jnp.concatenate([sw, sw, sw, sw], axis=1) * maskP).astype(bf)

        st = jax.lax.dot_general(swm, fx_h, (((0,), (0,)), ((), ())),
                                 preferred_element_type=f32)
        sn = jax.lax.dot_general(swm, ones_col, (((0,), (0,)), ((), ())),
                                 preferred_element_type=f32)
        stok = st / (sn + 1e-5)

        q = jnp.dot(stok, Wq_ref[:, :], preferred_element_type=f32) + bq_ref[:, :]
        k = jnp.dot(stok, Wk_ref[:, :], preferred_element_type=f32) + bk_ref[:, :]
        v = jnp.dot(stok, Wv_ref[:, :], preferred_element_type=f32) + bv_ref[:, :]
        k_son = jnp.dot(son[:, c0:c1], Wck_ref[:, :], preferred_element_type=f32) + bck_ref[:, :]
        v_son = jnp.dot(son[:, c0:c1], Wcv_ref[:, :], preferred_element_type=f32) + bcv_ref[:, :]

        blocks = []
        for b in range(NB):
            r0, r1 = b * SLICE_NUM, (b + 1) * SLICE_NUM
            qb, kb, vb = q[r0:r1], k[r0:r1], v[r0:r1]
            dots = jax.lax.dot_general(qb, kb, (((1,), (1,)), ((), ())),
                                       preferred_element_type=f32) * SCALE
            attn = _softmax_rows(dots)
            osb = jnp.dot(attn, vb, preferred_element_type=f32)

            m_row = (sbi_row == b).astype(f32)
            scores = jax.lax.dot_general(osb, k_son, (((1,), (1,)), ((), ())),
                                         preferred_element_type=f32) * SCALE
            scores = scores - (1.0 - m_row) * 1e9
            aw = _softmax_rows(scores)
            cross = jnp.dot(aw, v_son, preferred_element_type=f32)
            nonempty = (jnp.sum(m_row, keepdims=True) > 0.0).astype(f32)
            blocks.append(osb + cross * nonempty)
        full_os = jnp.concatenate(blocks, axis=0).astype(bf)

        outx_cols.append(jnp.dot(swm, full_os, preferred_element_type=f32).astype(bf))

    out_x = jnp.concatenate(outx_cols, axis=1)
    out_ref[:, :] = jnp.dot(out_x, Wout_ref[:, :].astype(bf),
                            preferred_element_type=f32)


@jax.jit
def kernel(x, batch_indices, sonata_features, sonata_batch_indices, temperature,
           W_x, b_x, W_fx, b_fx, W_slice, b_slice, W_q, b_q, W_k, b_k,
           W_v, b_v, W_ck, b_ck, W_cv, b_cv, W_out):
    inv_t = 1.0 / temperature.reshape(HEADS)
    eye_h = jnp.eye(HEADS, dtype=jnp.float32)
    Wslbd = (eye_h[:, None, :, None] * (W_slice[None, :, None, :] * inv_t[None, None, :, None])
             ).reshape(INNER, HG)
    bsl = (b_slice[None, :] * inv_t[:, None]).reshape(1, HG)
    bi = batch_indices.reshape(N_TOTAL, 1).astype(jnp.int32)
    sbi = sonata_batch_indices.reshape(1, M_TOTAL).astype(jnp.int32)
    return pl.pallas_call(
        _body,
        out_shape=jax.ShapeDtypeStruct((N_TOTAL, DIM), jnp.float32),
    )(x, bi, sonata_features, sbi,
      W_x, b_x.reshape(1, INNER), Wslbd, bsl, W_fx, b_fx.reshape(1, INNER),
      W_q, b_q.reshape(1, DIM_HEAD), W_k, b_k.reshape(1, DIM_HEAD),
      W_v, b_v.reshape(1, DIM_HEAD), W_ck, b_ck.reshape(1, DIM_HEAD),
      W_cv, b_cv.reshape(1, DIM_HEAD), W_out)


# speedup vs baseline: 27.2458x; 1.5490x over previous
import jax
import jax.numpy as jnp
from jax.experimental import pallas as pl

DIM = 256
HEADS = 8
DIM_HEAD = 64
SLICE_NUM = 32
INNER = HEADS * DIM_HEAD
N_TOTAL = 4096
M_TOTAL = 1024
NB = 4
SCALE = DIM_HEAD ** (-0.5)
BG = NB * SLICE_NUM
HG = HEADS * SLICE_NUM


def _softmax_rows(z):
    z = z - jnp.max(z, axis=1, keepdims=True)
    e = jnp.exp(z)
    return e / jnp.sum(e, axis=1, keepdims=True)


def _body(x_ref, bi_ref, son_ref, sbi_ref,
          Wx_ref, bx_ref, Wslbd_ref, bsl_ref, Wfx_ref, bfx_ref,
          Wqkv_ref, bqkv_ref, Wckbd_ref, bck_ref, Wcvbd_ref, bcv_ref,
          Wout_ref, out_ref):
    f32 = jnp.float32
    bf = jnp.bfloat16
    xb = x_ref[:, :].astype(bf)

    Wxsl = jnp.dot(Wx_ref[:, :], Wslbd_ref[:, :], preferred_element_type=f32)
    bxsl = jnp.dot(bx_ref[:, :], Wslbd_ref[:, :], preferred_element_type=f32) + bsl_ref[:, :]
    Wbig = jnp.concatenate([Wfx_ref[:, :], Wxsl], axis=1).astype(bf)
    bbig = jnp.concatenate([bfx_ref[:, :], bxsl], axis=1)
    big = jnp.dot(xb, Wbig, preferred_element_type=f32) + bbig
    fxm = big[:, :INNER].astype(bf)
    logits_all = big[:, INNER:]

    bi = bi_ref[:, :]
    blk = jax.lax.broadcasted_iota(jnp.int32, (N_TOTAL, BG), 1) // SLICE_NUM
    maskP = (blk == bi).astype(f32)

    rowb = jax.lax.broadcasted_iota(jnp.int32, (BG, BG), 0) // SLICE_NUM
    colb = jax.lax.broadcasted_iota(jnp.int32, (BG, BG), 1) // SLICE_NUM
    pen_bd = (rowb != colb).astype(f32) * 1e9

    sbi_row = sbi_ref[:, :]
    rb2 = jax.lax.broadcasted_iota(jnp.int32, (BG, M_TOTAL), 0) // SLICE_NUM
    mask2 = (rb2 == sbi_row).astype(f32)
    pen2 = (1.0 - mask2) * 1e9
    gate = (jnp.sum(mask2, axis=1, keepdims=True) > 0.0).astype(f32)

    sonb = son_ref[:, :].astype(bf)
    kson_all = (jnp.dot(sonb, Wckbd_ref[:, :].astype(bf), preferred_element_type=f32)
                + bck_ref[:, :]).astype(bf)
    vson_all = (jnp.dot(sonb, Wcvbd_ref[:, :].astype(bf), preferred_element_type=f32)
                + bcv_ref[:, :]).astype(bf)

    ones_col = jnp.ones((N_TOTAL, 1), bf)
    outx_cols = []
    for h in range(HEADS):
        c0, c1 = h * DIM_HEAD, (h + 1) * DIM_HEAD
        fx_h = fxm[:, c0:c1]
        sw = _softmax_rows(logits_all[:, h * SLICE_NUM:(h + 1) * SLICE_NUM])
        swm = (jnp.concatenate([sw, sw, sw, sw], axis=1) * maskP).astype(bf)

        st = jax.lax.dot_general(swm, fx_h, (((0,), (0,)), ((), ())),
                                 preferred_element_type=f32)
        sn = jax.lax.dot_general(swm, ones_col, (((0,), (0,)), ((), ())),
                                 preferred_element_type=f32)
        stok = st / (sn + 1e-5)

        qkv = jnp.dot(stok, Wqkv_ref[:, :], preferred_element_type=f32) + bqkv_ref[:, :]
        q, k, v = qkv[:, :DIM_HEAD], qkv[:, DIM_HEAD:2 * DIM_HEAD], qkv[:, 2 * DIM_HEAD:]

        dots = jax.lax.dot_general(q, k, (((1,), (1,)), ((), ())),
                                   preferred_element_type=f32) * SCALE - pen_bd
        osb = jnp.dot(_softmax_rows(dots), v, preferred_element_type=f32)

        scores = jax.lax.dot_general(osb.astype(bf), kson_all[:, c0:c1],
                                     (((1,), (1,)), ((), ())),
                                     preferred_element_type=f32) * SCALE - pen2
        aw = _softmax_rows(scores).astype(bf)
        cross = jnp.dot(aw, vson_all[:, c0:c1], preferred_element_type=f32)
        full_os = (osb + cross * gate).astype(bf)

        outx_cols.append(jnp.dot(swm, full_os, preferred_element_type=f32).astype(bf))

    out_x = jnp.concatenate(outx_cols, axis=1)
    out_ref[:, :] = jnp.dot(out_x, Wout_ref[:, :].astype(bf),
                            preferred_element_type=f32)


def _blockdiag(W, scale=None):
    eye_h = jnp.eye(HEADS, dtype=jnp.float32)
    out = eye_h[:, None, :, None] * W[None, :, None, :]
    if scale is not None:
        out = out * scale[None, None, :, None]
    return out.reshape(HEADS * W.shape[0], HEADS * W.shape[1])


@jax.jit
def kernel(x, batch_indices, sonata_features, sonata_batch_indices, temperature,
           W_x, b_x, W_fx, b_fx, W_slice, b_slice, W_q, b_q, W_k, b_k,
           W_v, b_v, W_ck, b_ck, W_cv, b_cv, W_out):
    inv_t = 1.0 / temperature.reshape(HEADS)
    Wslbd = _blockdiag(W_slice, inv_t)
    bsl = (b_slice[None, :] * inv_t[:, None]).reshape(1, HG)
    Wckbd = _blockdiag(W_ck)
    Wcvbd = _blockdiag(W_cv)
    bck = jnp.tile(b_ck, HEADS).reshape(1, INNER)
    bcv = jnp.tile(b_cv, HEADS).reshape(1, INNER)
    Wqkv = jnp.concatenate([W_q, W_k, W_v], axis=1)
    bqkv = jnp.concatenate([b_q, b_k, b_v]).reshape(1, 3 * DIM_HEAD)
    bi = batch_indices.reshape(N_TOTAL, 1).astype(jnp.int32)
    sbi = sonata_batch_indices.reshape(1, M_TOTAL).astype(jnp.int32)
    return pl.pallas_call(
        _body,
        out_shape=jax.ShapeDtypeStruct((N_TOTAL, DIM), jnp.float32),
    )(x, bi, sonata_features, sbi,
      W_x, b_x.reshape(1, INNER), Wslbd, bsl, W_fx, b_fx.reshape(1, INNER),
      Wqkv, bqkv, Wckbd, bck, Wcvbd, bcv, W_out)


# speedup vs baseline: 42.6098x; 1.5639x over previous
import jax
import jax.numpy as jnp
from jax.experimental import pallas as pl

DIM = 256
HEADS = 8
DIM_HEAD = 64
SLICE_NUM = 32
INNER = HEADS * DIM_HEAD
N_TOTAL = 4096
M_TOTAL = 1024
NB = 4
SCALE = DIM_HEAD ** (-0.5)
BG = NB * SLICE_NUM
HG = HEADS * SLICE_NUM


def _softmax_rows(z):
    z = z - jnp.max(z, axis=1, keepdims=True)
    e = jnp.exp(z)
    return e / jnp.sum(e, axis=1, keepdims=True)


def _body(x_ref, bi_ref, son_ref, sbi_ref,
          Wx_ref, bx_ref, Wslbd_ref, bsl_ref, Wfx_ref, bfx_ref,
          Wqkv_ref, bqkv_ref, Wckcv_ref, bckcv_ref,
          Wout_ref, out_ref):
    f32 = jnp.float32
    bf = jnp.bfloat16
    xb = x_ref[:, :].astype(bf)

    Wxsl = jnp.dot(Wx_ref[:, :], Wslbd_ref[:, :], preferred_element_type=f32)
    bxsl = jnp.dot(bx_ref[:, :], Wslbd_ref[:, :], preferred_element_type=f32) + bsl_ref[:, :]
    Wbig = jnp.concatenate([Wfx_ref[:, :], Wxsl], axis=1).astype(bf)
    bbig = jnp.concatenate([bfx_ref[:, :], bxsl], axis=1)
    big = jnp.dot(xb, Wbig, preferred_element_type=f32) + bbig
    fxm = big[:, :INNER].astype(bf)
    logits_all = big[:, INNER:]

    e_all = jnp.exp(logits_all)
    colh = jax.lax.broadcasted_iota(jnp.int32, (HG, HEADS), 0) // SLICE_NUM
    ones_blk = (colh == jax.lax.broadcasted_iota(jnp.int32, (HG, HEADS), 1)).astype(f32)
    denom = jnp.dot(jnp.dot(e_all, ones_blk, preferred_element_type=f32),
                    ones_blk.T, preferred_element_type=f32)
    sw_all = (e_all / denom).astype(bf)

    bi = bi_ref[:, :]
    blk = jax.lax.broadcasted_iota(jnp.int32, (N_TOTAL, BG), 1) // SLICE_NUM
    maskP = (blk == bi).astype(bf)

    rowb = jax.lax.broadcasted_iota(jnp.int32, (BG, BG), 0) // SLICE_NUM
    colb = jax.lax.broadcasted_iota(jnp.int32, (BG, BG), 1) // SLICE_NUM
    pen_bd = (rowb != colb).astype(f32) * 1e9

    sbi_row = sbi_ref[:, :]
    rb2 = jax.lax.broadcasted_iota(jnp.int32, (BG, M_TOTAL), 0) // SLICE_NUM
    mask2 = (rb2 == sbi_row).astype(f32)
    pen2 = (1.0 - mask2) * 1e9
    gate = (jnp.sum(mask2, axis=1, keepdims=True) > 0.0).astype(f32)

    sonb = son_ref[:, :].astype(bf)
    kvson = (jnp.dot(sonb, Wckcv_ref[:, :].astype(bf), preferred_element_type=f32)
             + bckcv_ref[:, :]).astype(bf)
    kson_all = kvson[:, :INNER]
    vson_all = kvson[:, INNER:]

    ones_col = jnp.ones((N_TOTAL, 1), bf)
    outx_cols = []
    for h in range(HEADS):
        c0, c1 = h * DIM_HEAD, (h + 1) * DIM_HEAD
        fx_h = fxm[:, c0:c1]
        sw = sw_all[:, h * SLICE_NUM:(h + 1) * SLICE_NUM]
        swm = jnp.concatenate([sw, sw, sw, sw], axis=1) * maskP

        st = jax.lax.dot_general(swm, fx_h, (((0,), (0,)), ((), ())),
                                 preferred_element_type=f32)
        sn = jax.lax.dot_general(swm, ones_col, (((0,), (0,)), ((), ())),
                                 preferred_element_type=f32)
        stok = st / (sn + 1e-5)

        qkv = jnp.dot(stok, Wqkv_ref[:, :], preferred_element_type=f32) + bqkv_ref[:, :]
        q, k, v = qkv[:, :DIM_HEAD], qkv[:, DIM_HEAD:2 * DIM_HEAD], qkv[:, 2 * DIM_HEAD:]

        dots = jax.lax.dot_general(q, k, (((1,), (1,)), ((), ())),
                                   preferred_element_type=f32) * SCALE - pen_bd
        osb = jnp.dot(_softmax_rows(dots), v, preferred_element_type=f32)

        scores = jax.lax.dot_general(osb.astype(bf), kson_all[:, c0:c1],
                                     (((1,), (1,)), ((), ())),
                                     preferred_element_type=f32) * SCALE - pen2
        aw = _softmax_rows(scores).astype(bf)
        cross = jnp.dot(aw, vson_all[:, c0:c1], preferred_element_type=f32)
        full_os = (osb + cross * gate).astype(bf)

        outx_cols.append(jnp.dot(swm, full_os, preferred_element_type=f32).astype(bf))

    out_x = jnp.concatenate(outx_cols, axis=1)
    out_ref[:, :] = jnp.dot(out_x, Wout_ref[:, :].astype(bf),
                            preferred_element_type=f32)


def _blockdiag(W, scale=None):
    eye_h = jnp.eye(HEADS, dtype=jnp.float32)
    out = eye_h[:, None, :, None] * W[None, :, None, :]
    if scale is not None:
        out = out * scale[None, None, :, None]
    return out.reshape(HEADS * W.shape[0], HEADS * W.shape[1])


@jax.jit
def kernel(x, batch_indices, sonata_features, sonata_batch_indices, temperature,
           W_x, b_x, W_fx, b_fx, W_slice, b_slice, W_q, b_q, W_k, b_k,
           W_v, b_v, W_ck, b_ck, W_cv, b_cv, W_out):
    inv_t = 1.0 / temperature.reshape(HEADS)
    Wslbd = _blockdiag(W_slice, inv_t)
    bsl = (b_slice[None, :] * inv_t[:, None]).reshape(1, HG)
    Wckcv = jnp.concatenate([_blockdiag(W_ck), _blockdiag(W_cv)], axis=1)
    bckcv = jnp.concatenate([jnp.tile(b_ck, HEADS), jnp.tile(b_cv, HEADS)]).reshape(1, 2 * INNER)
    Wqkv = jnp.concatenate([W_q, W_k, W_v], axis=1)
    bqkv = jnp.concatenate([b_q, b_k, b_v]).reshape(1, 3 * DIM_HEAD)
    bi = batch_indices.reshape(N_TOTAL, 1).astype(jnp.int32)
    sbi = sonata_batch_indices.reshape(1, M_TOTAL).astype(jnp.int32)
    return pl.pallas_call(
        _body,
        out_shape=jax.ShapeDtypeStruct((N_TOTAL, DIM), jnp.float32),
    )(x, bi, sonata_features, sbi,
      W_x, b_x.reshape(1, INNER), Wslbd, bsl, W_fx, b_fx.reshape(1, INNER),
      Wqkv, bqkv, Wckcv, bckcv, W_out)


# speedup vs baseline: 45.8654x; 1.0764x over previous
import jax
import jax.numpy as jnp
from jax.experimental import pallas as pl

DIM = 256
HEADS = 8
DIM_HEAD = 64
SLICE_NUM = 32
INNER = HEADS * DIM_HEAD
N_TOTAL = 4096
M_TOTAL = 1024
NB = 4
SCALE = DIM_HEAD ** (-0.5)
BG = NB * SLICE_NUM
HG = HEADS * SLICE_NUM


def _softmax_rows(z):
    z = z - jnp.max(z, axis=1, keepdims=True)
    e = jnp.exp(z)
    return e / jnp.sum(e, axis=1, keepdims=True)


def _body(x_ref, bi_ref, son_ref, sbi_ref,
          Wx_ref, bx_ref, Wslbd_ref, bsl_ref, Wfx_ref, bfx_ref,
          Wqkv_ref, bqkv_ref, Wckcv_ref, bckcv_ref,
          Wout_ref, out_ref):
    f32 = jnp.float32
    bf = jnp.bfloat16
    xb = x_ref[:, :].astype(bf)

    Wxsl = jnp.dot(Wx_ref[:, :], Wslbd_ref[:, :], preferred_element_type=f32)
    bxsl = jnp.dot(bx_ref[:, :], Wslbd_ref[:, :], preferred_element_type=f32) + bsl_ref[:, :]
    Wbig = jnp.concatenate([Wfx_ref[:, :], Wxsl], axis=1).astype(bf)
    bbig = jnp.concatenate([bfx_ref[:, :], bxsl], axis=1)
    big = jnp.dot(xb, Wbig, preferred_element_type=f32) + bbig
    fxm = big[:, :INNER].astype(bf)
    logits_all = big[:, INNER:]

    e_all = jnp.exp(logits_all)
    colh = jax.lax.broadcasted_iota(jnp.int32, (HG, HEADS), 0) // SLICE_NUM
    ones_blk = (colh == jax.lax.broadcasted_iota(jnp.int32, (HG, HEADS), 1)).astype(f32)
    denom = jnp.dot(jnp.dot(e_all, ones_blk, preferred_element_type=f32),
                    ones_blk.T, preferred_element_type=f32)
    sw_all = (e_all / denom).astype(bf)

    bi = bi_ref[:, :]
    blk = jax.lax.broadcasted_iota(jnp.int32, (N_TOTAL, BG), 1) // SLICE_NUM
    maskP = (blk == bi).astype(bf)

    rowb = jax.lax.broadcasted_iota(jnp.int32, (BG, BG), 0) // SLICE_NUM
    colb = jax.lax.broadcasted_iota(jnp.int32, (BG, BG), 1) // SLICE_NUM
    pen_bd = (rowb != colb).astype(f32) * 1e9

    sbi_row = sbi_ref[:, :]
    rb2 = jax.lax.broadcasted_iota(jnp.int32, (BG, M_TOTAL), 0) // SLICE_NUM
    mask2 = (rb2 == sbi_row).astype(f32)
    pen2 = (1.0 - mask2) * 1e9
    gate = (jnp.sum(mask2, axis=1, keepdims=True) > 0.0).astype(f32)

    sonb = son_ref[:, :].astype(bf)
    Wckcv = Wckcv_ref[:, :].astype(bf)
    bckcv = bckcv_ref[:, :]
    kvson = [
        (jnp.dot(sonb[:, h * DIM_HEAD:(h + 1) * DIM_HEAD], Wckcv,
                 preferred_element_type=f32) + bckcv).astype(bf)
        for h in range(HEADS)
    ]

    ones_col = jnp.ones((N_TOTAL, 1), bf)
    outx_cols = []
    for h in range(HEADS):
        c0, c1 = h * DIM_HEAD, (h + 1) * DIM_HEAD
        fx_h = fxm[:, c0:c1]
        sw = sw_all[:, h * SLICE_NUM:(h + 1) * SLICE_NUM]
        swm = jnp.concatenate([sw, sw, sw, sw], axis=1) * maskP

        st = jax.lax.dot_general(swm, fx_h, (((0,), (0,)), ((), ())),
                                 preferred_element_type=f32)
        sn = jax.lax.dot_general(swm, ones_col, (((0,), (0,)), ((), ())),
                                 preferred_element_type=f32)
        stok = st / (sn + 1e-5)

        qkv = jnp.dot(stok, Wqkv_ref[:, :], preferred_element_type=f32) + bqkv_ref[:, :]
        q, k, v = qkv[:, :DIM_HEAD], qkv[:, DIM_HEAD:2 * DIM_HEAD], qkv[:, 2 * DIM_HEAD:]

        dots = jax.lax.dot_general(q, k, (((1,), (1,)), ((), ())),
                                   preferred_element_type=f32) * SCALE - pen_bd
        osb = jnp.dot(_softmax_rows(dots), v, preferred_element_type=f32)

        scores = jax.lax.dot_general(osb.astype(bf), kvson[h][:, :DIM_HEAD],
                                     (((1,), (1,)), ((), ())),
                                     preferred_element_type=f32) * SCALE - pen2
        aw = _softmax_rows(scores).astype(bf)
        cross = jnp.dot(aw, kvson[h][:, DIM_HEAD:], preferred_element_type=f32)
        full_os = (osb + cross * gate).astype(bf)

        outx_cols.append(jnp.dot(swm, full_os, preferred_element_type=f32).astype(bf))

    out_x = jnp.concatenate(outx_cols, axis=1)
    out_ref[:, :] = jnp.dot(out_x, Wout_ref[:, :].astype(bf),
                            preferred_element_type=f32)


def _blockdiag(W, scale=None):
    eye_h = jnp.eye(HEADS, dtype=jnp.float32)
    out = eye_h[:, None, :, None] * W[None, :, None, :]
    if scale is not None:
        out = out * scale[None, None, :, None]
    return out.reshape(HEADS * W.shape[0], HEADS * W.shape[1])


@jax.jit
def kernel(x, batch_indices, sonata_features, sonata_batch_indices, temperature,
           W_x, b_x, W_fx, b_fx, W_slice, b_slice, W_q, b_q, W_k, b_k,
           W_v, b_v, W_ck, b_ck, W_cv, b_cv, W_out):
    inv_t = 1.0 / temperature.reshape(HEADS)
    Wslbd = _blockdiag(W_slice, inv_t)
    bsl = (b_slice[None, :] * inv_t[:, None]).reshape(1, HG)
    Wckcv = jnp.concatenate([W_ck, W_cv], axis=1)
    bckcv = jnp.concatenate([b_ck, b_cv]).reshape(1, 2 * DIM_HEAD)
    Wqkv = jnp.concatenate([W_q, W_k, W_v], axis=1)
    bqkv = jnp.concatenate([b_q, b_k, b_v]).reshape(1, 3 * DIM_HEAD)
    bi = batch_indices.reshape(N_TOTAL, 1).astype(jnp.int32)
    sbi = sonata_batch_indices.reshape(1, M_TOTAL).astype(jnp.int32)
    return pl.pallas_call(
        _body,
        out_shape=jax.ShapeDtypeStruct((N_TOTAL, DIM), jnp.float32),
    )(x, bi, sonata_features, sbi,
      W_x, b_x.reshape(1, INNER), Wslbd, bsl, W_fx, b_fx.reshape(1, INNER),
      Wqkv, bqkv, Wckcv, bckcv, W_out)


# speedup vs baseline: 47.5592x; 1.0369x over previous
import jax
import jax.numpy as jnp
from jax.experimental import pallas as pl

DIM = 256
HEADS = 8
DIM_HEAD = 64
SLICE_NUM = 32
INNER = HEADS * DIM_HEAD
N_TOTAL = 4096
M_TOTAL = 1024
NB = 4
SCALE = DIM_HEAD ** (-0.5)
BG = NB * SLICE_NUM
HG = HEADS * SLICE_NUM


def _body(x_ref, bi_ref, son_ref, sbi_ref,
          Wx_ref, bx_ref, Wslbd_ref, bsl_ref, Wfx_ref, bfx_ref,
          Wqkv_ref, bqkv_ref, Wckcv_ref, bckcv_ref,
          Wout_ref, out_ref):
    f32 = jnp.float32
    bf = jnp.bfloat16
    xb = x_ref[:, :].astype(bf)

    Wxsl = jnp.dot(Wx_ref[:, :], Wslbd_ref[:, :], preferred_element_type=f32)
    bxsl = jnp.dot(bx_ref[:, :], Wslbd_ref[:, :], preferred_element_type=f32) + bsl_ref[:, :]
    Wbig = jnp.concatenate([Wfx_ref[:, :], Wxsl], axis=1).astype(bf)
    bbig = jnp.concatenate([bfx_ref[:, :], bxsl], axis=1)
    big = jnp.dot(xb, Wbig, preferred_element_type=f32) + bbig
    fxm = big[:, :INNER].astype(bf)
    logits_all = big[:, INNER:]

    e_all = jnp.exp(logits_all)
    colh = jax.lax.broadcasted_iota(jnp.int32, (HG, HEADS), 0) // SLICE_NUM
    ones_blk = (colh == jax.lax.broadcasted_iota(jnp.int32, (HG, HEADS), 1)).astype(f32)
    denom = jnp.dot(jnp.dot(e_all, ones_blk, preferred_element_type=f32),
                    ones_blk.T, preferred_element_type=f32)
    sw_all = (e_all / denom).astype(bf)

    bi = bi_ref[:, :]
    blk = jax.lax.broadcasted_iota(jnp.int32, (N_TOTAL, BG), 1) // SLICE_NUM
    maskP = (blk == bi).astype(bf)

    rowb = jax.lax.broadcasted_iota(jnp.int32, (BG, BG), 0) // SLICE_NUM
    colb = jax.lax.broadcasted_iota(jnp.int32, (BG, BG), 1) // SLICE_NUM
    pen_bd = (rowb != colb).astype(f32) * 1e9

    sbi_row = sbi_ref[:, :]
    rb2 = jax.lax.broadcasted_iota(jnp.int32, (BG, M_TOTAL), 0) // SLICE_NUM
    pen2 = (rb2 != sbi_row).astype(f32) * 1e9

    sonb = son_ref[:, :].astype(bf)
    Wckcv = Wckcv_ref[:, :].astype(bf)
    bckcv = bckcv_ref[:, :]
    kvson = [
        (jnp.dot(sonb[:, h * DIM_HEAD:(h + 1) * DIM_HEAD], Wckcv,
                 preferred_element_type=f32) + bckcv).astype(bf)
        for h in range(HEADS)
    ]

    ones_col = jnp.ones((N_TOTAL, 1), bf)
    ones_bg = jnp.ones((BG, 1), f32)
    ones_m = jnp.ones((M_TOTAL, 1), bf)
    outx_cols = []
    for h in range(HEADS):
        c0, c1 = h * DIM_HEAD, (h + 1) * DIM_HEAD
        fx_h = fxm[:, c0:c1]
        sw = sw_all[:, h * SLICE_NUM:(h + 1) * SLICE_NUM]
        swm = jnp.concatenate([sw, sw, sw, sw], axis=1) * maskP

        st = jax.lax.dot_general(swm, fx_h, (((0,), (0,)), ((), ())),
                                 preferred_element_type=f32)
        sn = jax.lax.dot_general(swm, ones_col, (((0,), (0,)), ((), ())),
                                 preferred_element_type=f32)
        stok = st / (sn + 1e-5)

        qkv = jnp.dot(stok, Wqkv_ref[:, :], preferred_element_type=f32) + bqkv_ref[:, :]
        q, k, v = qkv[:, :DIM_HEAD], qkv[:, DIM_HEAD:2 * DIM_HEAD], qkv[:, 2 * DIM_HEAD:]

        dots = jax.lax.dot_general(q, k, (((1,), (1,)), ((), ())),
                                   preferred_element_type=f32) * SCALE - pen_bd
        de = jnp.exp(dots)
        rsd = jnp.dot(de, ones_bg, preferred_element_type=f32)
        osb = jnp.dot(de, v, preferred_element_type=f32) / rsd

        scores = jax.lax.dot_general(osb.astype(bf), kvson[h][:, :DIM_HEAD],
                                     (((1,), (1,)), ((), ())),
                                     preferred_element_type=f32) * SCALE - pen2
        se = jnp.exp(scores).astype(bf)
        rs = jnp.dot(se, ones_m, preferred_element_type=f32)
        cross = jnp.dot(se, kvson[h][:, DIM_HEAD:], preferred_element_type=f32)
        full_os = (osb + cross / (rs + 1e-30)).astype(bf)

        outx_cols.append(jnp.dot(swm, full_os, preferred_element_type=f32).astype(bf))

    out_x = jnp.concatenate(outx_cols, axis=1)
    out_ref[:, :] = jnp.dot(out_x, Wout_ref[:, :].astype(bf),
                            preferred_element_type=f32)


def _blockdiag(W, scale=None):
    eye_h = jnp.eye(HEADS, dtype=jnp.float32)
    out = eye_h[:, None, :, None] * W[None, :, None, :]
    if scale is not None:
        out = out * scale[None, None, :, None]
    return out.reshape(HEADS * W.shape[0], HEADS * W.shape[1])


@jax.jit
def kernel(x, batch_indices, sonata_features, sonata_batch_indices, temperature,
           W_x, b_x, W_fx, b_fx, W_slice, b_slice, W_q, b_q, W_k, b_k,
           W_v, b_v, W_ck, b_ck, W_cv, b_cv, W_out):
    inv_t = 1.0 / temperature.reshape(HEADS)
    Wslbd = _blockdiag(W_slice, inv_t)
    bsl = (b_slice[None, :] * inv_t[:, None]).reshape(1, HG)
    Wckcv = jnp.concatenate([W_ck, W_cv], axis=1)
    bckcv = jnp.concatenate([b_ck, b_cv]).reshape(1, 2 * DIM_HEAD)
    Wqkv = jnp.concatenate([W_q, W_k, W_v], axis=1)
    bqkv = jnp.concatenate([b_q, b_k, b_v]).reshape(1, 3 * DIM_HEAD)
    bi = batch_indices.reshape(N_TOTAL, 1).astype(jnp.int32)
    sbi = sonata_batch_indices.reshape(1, M_TOTAL).astype(jnp.int32)
    return pl.pallas_call(
        _body,
        out_shape=jax.ShapeDtypeStruct((N_TOTAL, DIM), jnp.float32),
    )(x, bi, sonata_features, sbi,
      W_x, b_x.reshape(1, INNER), Wslbd, bsl, W_fx, b_fx.reshape(1, INNER),
      Wqkv, bqkv, Wckcv, bckcv, W_out)
